# baseline (speedup 1.0000x reference)
import jax
import jax.numpy as jnp
from jax.experimental import pallas as pl

_RPB = 256


def _pos_body(t_ref, out_ref):
    hw = out_ref.shape[1]
    w2 = t_ref.shape[1]
    w = w2 // 2
    k_col = jax.lax.broadcasted_iota(jnp.int32, (w2, hw), 1)
    j_row = jax.lax.broadcasted_iota(jnp.int32, (w2, hw), 0)
    m = (((k_col % w) == j_row) | ((k_col // w + w) == j_row)).astype(
        jnp.float32
    )
    out_ref[...] = jnp.dot(t_ref[...], m, preferred_element_type=jnp.float32)


def kernel(x, row_embed, col_embed):
    b = x.shape[0]
    h, w = x.shape[-2], x.shape[-1]
    d = row_embed.shape[1]
    col_t = col_embed[:w].T
    row_t = row_embed[:h].T
    z_cw = jnp.zeros((d, h), col_t.dtype)
    z_rh = jnp.zeros((d, w), row_t.dtype)
    t = jnp.concatenate(
        [
            jnp.concatenate([col_t, z_cw], axis=1),
            jnp.concatenate([z_rh, row_t], axis=1),
        ],
        axis=0,
    )
    n_rows = b * 2 * d
    per_plane = (2 * d) // _RPB
    out = pl.pallas_call(
        _pos_body,
        grid=(n_rows // _RPB,),
        in_specs=[
            pl.BlockSpec((_RPB, w + h), lambda i: (i % per_plane, 0))
        ],
        out_specs=pl.BlockSpec((_RPB, h * w), lambda i: (i, 0)),
        out_shape=jax.ShapeDtypeStruct((n_rows, h * w), x.dtype),
    )(t)
    return out.reshape(b, 2 * d, h, w)


# speedup vs baseline: 1.0595x; 1.0595x over previous
import jax
import jax.numpy as jnp
from jax.experimental import pallas as pl

def _pos_body(t_ref, out_ref):
    d2, h, w = out_ref.shape[1], out_ref.shape[2], out_ref.shape[3]
    hw = h * w
    w2 = t_ref.shape[1]
    k_col = jax.lax.broadcasted_iota(jnp.int32, (w2, hw), 1)
    j_row = jax.lax.broadcasted_iota(jnp.int32, (w2, hw), 0)
    m = (((k_col % w) == j_row) | ((k_col // w + w) == j_row)).astype(
        jnp.float32
    )
    plane = jnp.dot(t_ref[...], m, preferred_element_type=jnp.float32)
    out_ref[0] = plane.reshape(d2, h, w)


def kernel(x, row_embed, col_embed):
    b = x.shape[0]
    h, w = x.shape[-2], x.shape[-1]
    d = row_embed.shape[1]
    col_t = col_embed[:w].T
    row_t = row_embed[:h].T
    z_cw = jnp.zeros((d, h), col_t.dtype)
    z_rh = jnp.zeros((d, w), row_t.dtype)
    t = jnp.concatenate(
        [
            jnp.concatenate([col_t, z_cw], axis=1),
            jnp.concatenate([z_rh, row_t], axis=1),
        ],
        axis=0,
    )
    out = pl.pallas_call(
        _pos_body,
        grid=(b,),
        in_specs=[pl.BlockSpec((2 * d, w + h), lambda i: (0, 0))],
        out_specs=pl.BlockSpec((1, 2 * d, h, w), lambda i: (i, 0, 0, 0)),
        out_shape=jax.ShapeDtypeStruct((b, 2 * d, h, w), x.dtype),
    )(t)
    return out


# speedup vs baseline: 1.1419x; 1.0777x over previous
import jax
import jax.numpy as jnp
from jax.experimental import pallas as pl
from jax.experimental.pallas import tpu as pltpu


def _pos_body(t_ref, out_ref, plane_ref, sems):
    d2, h, w = plane_ref.shape
    hw = h * w
    w2 = t_ref.shape[1]
    b = out_ref.shape[0]
    k_col = jax.lax.broadcasted_iota(jnp.int32, (w2, hw), 1)
    j_row = jax.lax.broadcasted_iota(jnp.int32, (w2, hw), 0)
    m = (((k_col % w) == j_row) | ((k_col // w + w) == j_row)).astype(
        jnp.float32
    )
    plane = jax.lax.dot(
        t_ref[...],
        m,
        precision=jax.lax.Precision.HIGHEST,
        preferred_element_type=jnp.float32,
    )
    plane_ref[...] = plane.reshape(d2, h, w)
    copies = [
        pltpu.make_async_copy(plane_ref, out_ref.at[q], sems.at[q])
        for q in range(b)
    ]
    for c in copies:
        c.start()
    for c in copies:
        c.wait()


def kernel(x, row_embed, col_embed):
    b = x.shape[0]
    h, w = x.shape[-2], x.shape[-1]
    d = row_embed.shape[1]
    col_t = col_embed[:w].T
    row_t = row_embed[:h].T
    z_cw = jnp.zeros((d, h), col_t.dtype)
    z_rh = jnp.zeros((d, w), row_t.dtype)
    t = jnp.concatenate(
        [
            jnp.concatenate([col_t, z_cw], axis=1),
            jnp.concatenate([z_rh, row_t], axis=1),
        ],
        axis=0,
    )
    return pl.pallas_call(
        _pos_body,
        in_specs=[pl.BlockSpec(memory_space=pltpu.VMEM)],
        out_specs=pl.BlockSpec(memory_space=pl.ANY),
        out_shape=jax.ShapeDtypeStruct((b, 2 * d, h, w), x.dtype),
        scratch_shapes=[
            pltpu.VMEM((2 * d, h, w), jnp.float32),
            pltpu.SemaphoreType.DMA((b,)),
        ],
    )(t)


# speedup vs baseline: 8.2407x; 7.2169x over previous
import jax
import jax.numpy as jnp
from jax.experimental import pallas as pl
from jax.experimental.pallas import tpu as pltpu


def _pos_body(col_ref, row_ref, out_ref, plane_ref, sems):
    h, w, d2 = plane_ref.shape
    d = d2 // 2
    b = out_ref.shape[0]
    plane_ref[:, :, :d] = jnp.broadcast_to(col_ref[...][None], (h, w, d))
    plane_ref[:, :, d:] = jnp.broadcast_to(row_ref[...][:, None, :], (h, w, d))
    copies = [
        pltpu.make_async_copy(plane_ref, out_ref.at[q], sems.at[q])
        for q in range(b)
    ]
    for c in copies:
        c.start()
    for c in copies:
        c.wait()


def kernel(x, row_embed, col_embed):
    b = x.shape[0]
    h, w = x.shape[-2], x.shape[-1]
    d = row_embed.shape[1]
    out = pl.pallas_call(
        _pos_body,
        in_specs=[
            pl.BlockSpec(memory_space=pltpu.VMEM),
            pl.BlockSpec(memory_space=pltpu.VMEM),
        ],
        out_specs=pl.BlockSpec(memory_space=pl.ANY),
        out_shape=jax.ShapeDtypeStruct((b, h, w, 2 * d), x.dtype),
        scratch_shapes=[
            pltpu.VMEM((h, w, 2 * d), jnp.float32),
            pltpu.SemaphoreType.DMA((b,)),
        ],
    )(col_embed[:w], row_embed[:h])
    return jnp.transpose(out, (0, 3, 1, 2))
